# baseline (speedup 1.0000x reference)
import numpy as np

import jax
import jax.numpy as jnp
from jax import lax
from jax.experimental import pallas as pl
from jax.experimental.pallas import tpu as pltpu

_NB = 512

_C1_ROWS = 588
_C1_BLK = 640
_C2_ROWS = 200
_C2_BLK = 256


def _pool_basis(out_size, in_size):
    v = np.zeros((2, 5, out_size, in_size), np.float32)
    for d in range(2):
        for i in range(5):
            for p in range(out_size):
                r = 2 * p + d + i
                if r < in_size:
                    v[d, i, p, r] = 1.0
    return v


_V1 = _pool_basis(14, 32)
_V2 = _pool_basis(5, 14)


def _bias_selector():
    sel = np.zeros((123, 1024), np.float32)
    for c in range(3):
        sel[c, c * 196:(c + 1) * 196] = 1.0
    for o in range(8):
        sel[3 + o, 640 + o * 25:640 + (o + 1) * 25] = 1.0
    for k in range(60):
        sel[11 + k, 896 + k] = 1.0
    for k in range(42):
        sel[71 + k, 960 + k] = 1.0
    for k in range(10):
        sel[113 + k, 1008 + k] = 1.0
    return sel


_BSEL = _bias_selector()


def _build_mats(conv1_w, conv1_b, conv2_w, conv2_b, conv3_w, conv3_b,
                fc1_w, fc1_b, fc2_w, fc2_b):
    f32 = jnp.float32
    v1 = jnp.asarray(_V1)
    v2 = jnp.asarray(_V2)

    w1 = conv1_w.astype(f32).reshape(3, 5, 5)
    h1 = jnp.einsum("cij,ejqs->iceqs", w1, v1).astype(jnp.bfloat16)
    m1 = jnp.einsum("dipr,iceqs->dprceqs",
                    v1.astype(jnp.bfloat16), h1)
    a1 = m1.transpose(2, 6, 0, 4, 3, 1, 5)
    a1 = a1.reshape(1024, 4, _C1_ROWS)
    a1 = jnp.pad(a1, ((0, 0), (0, 0), (0, _C1_BLK - _C1_ROWS)))
    a1 = a1.reshape(1024, 4 * _C1_BLK)

    w2 = conv2_w.astype(f32)
    h2 = jnp.einsum("ocij,ejqs->ioceqs", w2, v2).astype(jnp.bfloat16)
    m2 = jnp.einsum("dipr,ioceqs->dproceqs",
                    v2.astype(jnp.bfloat16), h2)
    a2 = m2.transpose(4, 2, 7, 0, 5, 3, 1, 6)
    a2 = a2.reshape(_C1_ROWS, 4, _C2_ROWS)
    a2 = jnp.pad(a2, ((0, _C1_BLK - _C1_ROWS), (0, 0), (0, _C2_BLK - _C2_ROWS)))
    a2 = a2.reshape(_C1_BLK, 4 * _C2_BLK)

    a3 = jnp.pad(conv3_w.astype(f32).reshape(60, 200),
                 ((0, 4), (0, _C2_BLK - 200)))
    af1 = jnp.pad(fc1_w.astype(f32), ((0, 6), (0, 4)))
    af2 = jnp.pad(fc2_w.astype(f32), ((0, 6), (0, 6)))

    bvec = jnp.concatenate([conv1_b.astype(f32), conv2_b.astype(f32),
                            conv3_b.astype(f32), fc1_b.astype(f32),
                            fc2_b.astype(f32)])
    ball = jnp.dot(bvec[None, :], jnp.asarray(_BSEL))
    return a1, a2, a3, af1, af2, ball


def _fwd_kernel(x_ref, a1_hbm, a2_hbm, a3_ref, af1_ref, af2_ref, ball_ref,
                logits_ref, feat_ref,
                a1_vm, a2_vm, sem):
    f32 = jnp.float32
    tdims = (((1,), (1,)), ((), ()))

    @pl.when(pl.program_id(0) == 0)
    def _():
        c1 = pltpu.make_async_copy(a1_hbm, a1_vm, sem.at[0])
        c2 = pltpu.make_async_copy(a2_hbm, a2_vm, sem.at[1])
        c1.start()
        c2.start()
        c1.wait()
        c2.wait()

    ball = ball_ref[...]
    b1 = ball[:, 0:_C1_BLK]
    b2 = ball[:, _C1_BLK:_C1_BLK + _C2_BLK]
    b3 = ball[:, 896:960]
    bf1 = ball[:, 960:1008]
    bf2 = ball[:, 1008:1024]

    x = x_ref[...].astype(jnp.bfloat16)

    y = jnp.dot(x, a1_vm[...], preferred_element_type=f32)
    m = jnp.maximum(jnp.maximum(y[:, 0:_C1_BLK], y[:, _C1_BLK:2 * _C1_BLK]),
                    jnp.maximum(y[:, 2 * _C1_BLK:3 * _C1_BLK],
                                y[:, 3 * _C1_BLK:4 * _C1_BLK]))
    p1 = jnp.maximum(m + b1, 0.0)

    y2 = jnp.dot(p1.astype(jnp.bfloat16), a2_vm[...],
                 preferred_element_type=f32)
    m2 = jnp.maximum(jnp.maximum(y2[:, 0:_C2_BLK], y2[:, _C2_BLK:2 * _C2_BLK]),
                     jnp.maximum(y2[:, 2 * _C2_BLK:3 * _C2_BLK],
                                 y2[:, 3 * _C2_BLK:4 * _C2_BLK]))
    p2 = jnp.maximum(m2 + b2, 0.0)

    feat = jnp.maximum(lax.dot_general(p2, a3_ref[...], tdims,
                                       preferred_element_type=f32)
                       + b3, 0.0)
    h = jnp.maximum(lax.dot_general(feat, af1_ref[...], tdims,
                                    preferred_element_type=f32)
                    + bf1, 0.0)
    logits_ref[...] = (lax.dot_general(h, af2_ref[...], tdims,
                                       preferred_element_type=f32)
                       + bf2)
    feat_ref[...] = feat


def kernel(conv1_w, conv1_b, conv2_w, conv2_b, conv3_w, conv3_b,
           fc1_w, fc1_b, fc2_w, fc2_b, img):
    n = img.shape[0]
    n_pad = ((n + _NB - 1) // _NB) * _NB
    x = img.astype(jnp.float32).reshape(n, 1024)
    if n_pad != n:
        x = jnp.pad(x, ((0, n_pad - n), (0, 0)))

    mats = _build_mats(conv1_w, conv1_b, conv2_w, conv2_b, conv3_w, conv3_b,
                       fc1_w, fc1_b, fc2_w, fc2_b)

    def fixed(shape):
        return pl.BlockSpec(shape, lambda b: (0,) * len(shape))

    hbm = pl.BlockSpec(memory_space=pltpu.MemorySpace.HBM)

    logits_p, feat_p = pl.pallas_call(
        _fwd_kernel,
        out_shape=(jax.ShapeDtypeStruct((n_pad, 16), jnp.float32),
                   jax.ShapeDtypeStruct((n_pad, 64), jnp.float32)),
        grid=(n_pad // _NB,),
        in_specs=[
            pl.BlockSpec((_NB, 1024), lambda b: (b, 0)),
            hbm,
            hbm,
            fixed((64, _C2_BLK)),
            fixed((48, 64)),
            fixed((16, 48)),
            fixed((1, 1024)),
        ],
        out_specs=(pl.BlockSpec((_NB, 16), lambda b: (b, 0)),
                   pl.BlockSpec((_NB, 64), lambda b: (b, 0))),
        scratch_shapes=[
            pltpu.VMEM((1024, 4 * _C1_BLK), jnp.bfloat16),
            pltpu.VMEM((_C1_BLK, 4 * _C2_BLK), jnp.bfloat16),
            pltpu.SemaphoreType.DMA((2,)),
        ],
        compiler_params=pltpu.CompilerParams(
            dimension_semantics=("arbitrary",)),
    )(x, *mats)

    return logits_p[:n, :10], feat_p[:n, :60]


# speedup vs baseline: 1.0121x; 1.0121x over previous
import numpy as np

import jax
import jax.numpy as jnp
from jax import lax
from jax.experimental import pallas as pl
from jax.experimental.pallas import tpu as pltpu

_NB = 1024

_C1_ROWS = 588
_C1_BLK = 640
_C2_ROWS = 200
_C2_BLK = 256


def _pool_basis(out_size, in_size):
    v = np.zeros((2, 5, out_size, in_size), np.float32)
    for d in range(2):
        for i in range(5):
            for p in range(out_size):
                r = 2 * p + d + i
                if r < in_size:
                    v[d, i, p, r] = 1.0
    return v


_V1 = _pool_basis(14, 32)
_V2 = _pool_basis(5, 14)


def _bias_selector():
    sel = np.zeros((123, 1024), np.float32)
    for c in range(3):
        sel[c, c * 196:(c + 1) * 196] = 1.0
    for o in range(8):
        sel[3 + o, 640 + o * 25:640 + (o + 1) * 25] = 1.0
    for k in range(60):
        sel[11 + k, 896 + k] = 1.0
    for k in range(42):
        sel[71 + k, 960 + k] = 1.0
    for k in range(10):
        sel[113 + k, 1008 + k] = 1.0
    return sel


_BSEL = _bias_selector()


def _build_mats(conv1_w, conv1_b, conv2_w, conv2_b, conv3_w, conv3_b,
                fc1_w, fc1_b, fc2_w, fc2_b):
    f32 = jnp.float32
    v1 = jnp.asarray(_V1)
    v2 = jnp.asarray(_V2)

    w1 = conv1_w.astype(f32).reshape(3, 5, 5)
    h1 = jnp.einsum("cij,ejqs->iceqs", w1, v1)
    m1 = jnp.einsum("dipr,iceqs->dprceqs", v1, h1)
    a1 = m1.astype(jnp.bfloat16).transpose(2, 6, 0, 4, 3, 1, 5)
    a1 = a1.reshape(1024, 4, _C1_ROWS)
    a1 = jnp.pad(a1, ((0, 0), (0, 0), (0, _C1_BLK - _C1_ROWS)))
    a1 = a1.reshape(1024, 4 * _C1_BLK)

    w2 = conv2_w.astype(f32)
    h2 = jnp.einsum("ocij,ejqs->ioceqs", w2, v2)
    m2 = jnp.einsum("dipr,ioceqs->dproceqs", v2, h2)
    a2 = m2.astype(jnp.bfloat16).transpose(4, 2, 7, 0, 5, 3, 1, 6)
    a2 = a2.reshape(_C1_ROWS, 4, _C2_ROWS)
    a2 = jnp.pad(a2, ((0, _C1_BLK - _C1_ROWS), (0, 0), (0, _C2_BLK - _C2_ROWS)))
    a2 = a2.reshape(_C1_BLK, 4 * _C2_BLK)

    a3 = jnp.pad(conv3_w.astype(f32).reshape(60, 200),
                 ((0, 4), (0, _C2_BLK - 200)))
    af1 = jnp.pad(fc1_w.astype(f32), ((0, 6), (0, 4)))
    af2 = jnp.pad(fc2_w.astype(f32), ((0, 6), (0, 6)))

    bvec = jnp.concatenate([conv1_b.astype(f32), conv2_b.astype(f32),
                            conv3_b.astype(f32), fc1_b.astype(f32),
                            fc2_b.astype(f32)])
    ball = jnp.dot(bvec[None, :], jnp.asarray(_BSEL))
    return a1, a2, a3, af1, af2, ball


def _fwd_kernel(x_ref, a1_hbm, a2_hbm, a3_ref, af1_ref, af2_ref, ball_ref,
                logits_ref, feat_ref,
                a1_vm, a2_vm, sem):
    f32 = jnp.float32
    tdims = (((1,), (1,)), ((), ()))

    @pl.when(pl.program_id(0) == 0)
    def _():
        c1 = pltpu.make_async_copy(a1_hbm, a1_vm, sem.at[0])
        c2 = pltpu.make_async_copy(a2_hbm, a2_vm, sem.at[1])
        c1.start()
        c2.start()
        c1.wait()
        c2.wait()

    ball = ball_ref[...]
    b1 = ball[:, 0:_C1_BLK]
    b2 = ball[:, _C1_BLK:_C1_BLK + _C2_BLK]
    b3 = ball[:, 896:960]
    bf1 = ball[:, 960:1008]
    bf2 = ball[:, 1008:1024]

    x = x_ref[...].astype(jnp.bfloat16)

    y = jnp.dot(x, a1_vm[...], preferred_element_type=f32)
    m = jnp.maximum(jnp.maximum(y[:, 0:_C1_BLK], y[:, _C1_BLK:2 * _C1_BLK]),
                    jnp.maximum(y[:, 2 * _C1_BLK:3 * _C1_BLK],
                                y[:, 3 * _C1_BLK:4 * _C1_BLK]))
    p1 = jnp.maximum(m + b1, 0.0)

    y2 = jnp.dot(p1.astype(jnp.bfloat16), a2_vm[...],
                 preferred_element_type=f32)
    m2 = jnp.maximum(jnp.maximum(y2[:, 0:_C2_BLK], y2[:, _C2_BLK:2 * _C2_BLK]),
                     jnp.maximum(y2[:, 2 * _C2_BLK:3 * _C2_BLK],
                                 y2[:, 3 * _C2_BLK:4 * _C2_BLK]))
    p2 = jnp.maximum(m2 + b2, 0.0)

    feat = jnp.maximum(lax.dot_general(p2, a3_ref[...], tdims,
                                       preferred_element_type=f32)
                       + b3, 0.0)
    h = jnp.maximum(lax.dot_general(feat, af1_ref[...], tdims,
                                    preferred_element_type=f32)
                    + bf1, 0.0)
    logits_ref[...] = (lax.dot_general(h, af2_ref[...], tdims,
                                       preferred_element_type=f32)
                       + bf2)
    feat_ref[...] = feat


def kernel(conv1_w, conv1_b, conv2_w, conv2_b, conv3_w, conv3_b,
           fc1_w, fc1_b, fc2_w, fc2_b, img):
    n = img.shape[0]
    n_pad = ((n + _NB - 1) // _NB) * _NB
    x = img.astype(jnp.float32).reshape(n, 1024)
    if n_pad != n:
        x = jnp.pad(x, ((0, n_pad - n), (0, 0)))

    mats = _build_mats(conv1_w, conv1_b, conv2_w, conv2_b, conv3_w, conv3_b,
                       fc1_w, fc1_b, fc2_w, fc2_b)

    def fixed(shape):
        return pl.BlockSpec(shape, lambda b: (0,) * len(shape))

    hbm = pl.BlockSpec(memory_space=pltpu.MemorySpace.HBM)

    logits_p, feat_p = pl.pallas_call(
        _fwd_kernel,
        out_shape=(jax.ShapeDtypeStruct((n_pad, 16), jnp.float32),
                   jax.ShapeDtypeStruct((n_pad, 64), jnp.float32)),
        grid=(n_pad // _NB,),
        in_specs=[
            pl.BlockSpec((_NB, 1024), lambda b: (b, 0)),
            hbm,
            hbm,
            fixed((64, _C2_BLK)),
            fixed((48, 64)),
            fixed((16, 48)),
            fixed((1, 1024)),
        ],
        out_specs=(pl.BlockSpec((_NB, 16), lambda b: (b, 0)),
                   pl.BlockSpec((_NB, 64), lambda b: (b, 0))),
        scratch_shapes=[
            pltpu.VMEM((1024, 4 * _C1_BLK), jnp.bfloat16),
            pltpu.VMEM((_C1_BLK, 4 * _C2_BLK), jnp.bfloat16),
            pltpu.SemaphoreType.DMA((2,)),
        ],
        compiler_params=pltpu.CompilerParams(
            dimension_semantics=("arbitrary",)),
    )(x, *mats)

    return logits_p[:n, :10], feat_p[:n, :60]
